# baseline (speedup 1.0000x reference)
import functools

import jax
import jax.numpy as jnp
from jax import lax
from jax.experimental import pallas as pl
from jax.experimental.pallas import tpu as pltpu
from jax.experimental.pallas import tpu_sc as plsc

N = 10000
NP = 10112
E = 320000
F_IN = 128
HID = 128
C = 40
CP = 48

NC = 2
NS = 16
NW = NC * NS
EPW = E // NW
CH = 96
GRP = 6
NG = 18
ITERS = NG * GRP
EPWP = ITERS * CH
RPT = NP // NS


def _sc_aggregate(D, with_deg, tc_tiling=True):
    out_type = [jax.ShapeDtypeStruct((NC, NP, D), jnp.float32)]
    scratch = [
        pltpu.VMEM((2, 2, GRP, CH), jnp.int32),
        pltpu.VMEM((3, CH, D), jnp.float32),
        pltpu.VMEM_SHARED((NP, D), jnp.float32),
        pltpu.SemaphoreType.DMA,
        pltpu.SemaphoreType.DMA,
        pltpu.SemaphoreType.DMA,
        pltpu.SemaphoreType.DMA,
        pltpu.SemaphoreType.DMA,
        pltpu.SemaphoreType.DMA,
        pltpu.SemaphoreType.DMA,
    ]
    if with_deg:
        out_type.append(jax.ShapeDtypeStruct((NC, NP), jnp.float32))
        scratch += [
            pltpu.VMEM((CH,), jnp.float32),
            pltpu.VMEM_SHARED((NP,), jnp.float32),
        ]

    mesh = plsc.VectorSubcoreMesh(
        core_axis_name="c", subcore_axis_name="s", num_cores=NC, num_subcores=NS
    )

    cp = pltpu.CompilerParams(use_tc_tiling_on_sc=tc_tiling)

    @functools.partial(pl.kernel, out_type=out_type, mesh=mesh, scratch_types=scratch,
                       compiler_params=cp)
    def agg(*refs):
        if with_deg:
            (table, srcw, dstw, zfeat, zdeg, onesh,
             part_out, deg_out,
             idxb, rows, acc, sr0, sr1, sr2, ss0, ss1, ss2, semi,
             ones_v, dacc) = refs
        else:
            (table, srcw, dstw, zfeat,
             part_out,
             idxb, rows, acc, sr0, sr1, sr2, ss0, ss1, ss2, semi) = refs

        c = lax.axis_index("c")
        s = lax.axis_index("s")
        w = s * NC + c
        semr = (sr0, sr1, sr2)
        sems = (ss0, ss1, ss2)

        def g_start(idx_ref, b):
            pltpu.async_copy(table.at[idx_ref], rows.at[b], semr[b])

        def g_wait(idx_ref, b):
            pltpu.make_async_copy(table.at[idx_ref], rows.at[b], semr[b]).wait()

        def s_start(idx_ref, b):
            pltpu.async_copy(rows.at[b], acc.at[idx_ref], sems[b], add=True)
            if with_deg:
                pltpu.async_copy(ones_v, dacc.at[idx_ref], sems[b], add=True)

        def s_wait(idx_ref, b):
            pltpu.make_async_copy(rows.at[b], acc.at[idx_ref], sems[b]).wait()
            if with_deg:
                pltpu.make_async_copy(ones_v, dacc.at[idx_ref], sems[b]).wait()

        blk0 = w * NG
        pltpu.sync_copy(srcw.at[blk0], idxb.at[0, 0])
        pltpu.sync_copy(dstw.at[blk0], idxb.at[0, 1])
        g_start(idxb.at[0, 0, 0], 0)
        g_start(idxb.at[0, 0, 1], 1)

        pltpu.sync_copy(zfeat.at[pl.ds(s * RPT, RPT)], acc.at[pl.ds(s * RPT, RPT)])
        if with_deg:
            pltpu.sync_copy(onesh, ones_v)

            @pl.when(s == 0)
            def _():
                pltpu.sync_copy(zdeg, dacc)

        plsc.subcore_barrier()

        def group(g, carry):
            half = lax.rem(g, 2)
            oh = 1 - half
            for j in range(GRP):
                b = j % 3
                bp = (j - 1) % 3
                g_wait(idxb.at[half, 0, j], b)
                s_start(idxb.at[half, 1, j], b)
                if j == 0:
                    @pl.when(g > 0)
                    def _():
                        s_wait(idxb.at[oh, 1, GRP - 1], bp)

                    @pl.when(g < NG - 1)
                    def _():
                        pltpu.async_copy(srcw.at[blk0 + g + 1], idxb.at[oh, 0], semi)
                        pltpu.async_copy(dstw.at[blk0 + g + 1], idxb.at[oh, 1], semi)
                else:
                    s_wait(idxb.at[half, 1, j - 1], bp)
                if j < GRP - 2:
                    g_start(idxb.at[half, 0, j + 2], bp)
                else:
                    if j == GRP - 2:
                        @pl.when(g < NG - 1)
                        def _():
                            pltpu.make_async_copy(srcw.at[blk0 + g + 1], idxb.at[oh, 0],
                                                  semi).wait()
                            pltpu.make_async_copy(dstw.at[blk0 + g + 1], idxb.at[oh, 1],
                                                  semi).wait()

                    @pl.when(g < NG - 1)
                    def _():
                        g_start(idxb.at[oh, 0, j + 2 - GRP], bp)
            return carry

        lax.fori_loop(0, NG, group, 0)
        s_wait(idxb.at[(NG - 1) % 2, 1, GRP - 1], (GRP - 1) % 3)
        plsc.subcore_barrier()

        pltpu.sync_copy(acc.at[pl.ds(s * RPT, RPT)], part_out.at[c, pl.ds(s * RPT, RPT)])
        if with_deg:
            @pl.when(s == 0)
            def _():
                pltpu.sync_copy(dacc, deg_out.at[c])

    return agg


def _tc_pre(x, wl, wr):
    B = 1264

    def body(x_ref, wl_ref, wr_ref, xt_ref, xr_ref):
        xb = x_ref[...]
        xt_ref[...] = jnp.dot(xb, wl_ref[...], preferred_element_type=jnp.float32)
        xr_ref[...] = jnp.dot(xb, wr_ref[...], preferred_element_type=jnp.float32)

    return pl.pallas_call(
        body,
        grid=(NP // B,),
        in_specs=[
            pl.BlockSpec((B, F_IN), lambda i: (i, 0)),
            pl.BlockSpec((F_IN, HID), lambda i: (0, 0)),
            pl.BlockSpec((F_IN, HID), lambda i: (0, 0)),
        ],
        out_specs=[pl.BlockSpec((B, HID), lambda i: (i, 0))] * 2,
        out_shape=[jax.ShapeDtypeStruct((NP, HID), jnp.float32)] * 2,
    )(x, wl, wr)


def _tc_mid(p0, p1, d0, d1, xr, b0, wl1, wr1, b1):
    B = 1264

    def body(p0_ref, p1_ref, d0_ref, d1_ref, xr_ref, b0_ref, wl_ref, wr_ref, b1_ref,
             ht_ref, hr_ref, inv_ref):
        inv = 1.0 / jnp.maximum(d0_ref[...] + d1_ref[...], 1.0)
        h = jnp.maximum((p0_ref[...] + p1_ref[...]) * inv + b0_ref[...] + xr_ref[...], 0.0)
        ht_ref[...] = jnp.dot(h, wl_ref[...], preferred_element_type=jnp.float32)
        hr_ref[...] = jnp.dot(h, wr_ref[...], preferred_element_type=jnp.float32) + b1_ref[...]
        inv_ref[...] = inv

    return pl.pallas_call(
        body,
        grid=(NP // B,),
        in_specs=[
            pl.BlockSpec((B, HID), lambda i: (i, 0)),
            pl.BlockSpec((B, HID), lambda i: (i, 0)),
            pl.BlockSpec((B, 1), lambda i: (i, 0)),
            pl.BlockSpec((B, 1), lambda i: (i, 0)),
            pl.BlockSpec((B, HID), lambda i: (i, 0)),
            pl.BlockSpec((1, HID), lambda i: (0, 0)),
            pl.BlockSpec((HID, CP), lambda i: (0, 0)),
            pl.BlockSpec((HID, CP), lambda i: (0, 0)),
            pl.BlockSpec((1, CP), lambda i: (0, 0)),
        ],
        out_specs=[
            pl.BlockSpec((B, CP), lambda i: (i, 0)),
            pl.BlockSpec((B, CP), lambda i: (i, 0)),
            pl.BlockSpec((B, 1), lambda i: (i, 0)),
        ],
        out_shape=[
            jax.ShapeDtypeStruct((NP, CP), jnp.float32),
            jax.ShapeDtypeStruct((NP, CP), jnp.float32),
            jax.ShapeDtypeStruct((NP, 1), jnp.float32),
        ],
    )(p0, p1, d0, d1, xr, b0, wl1, wr1, b1)


def _tc_post(q0, q1, inv, hr):
    B = 1264

    def body(q0_ref, q1_ref, inv_ref, hr_ref, o_ref):
        o_ref[...] = (q0_ref[...] + q1_ref[...]) * inv_ref[...] + hr_ref[...]

    return pl.pallas_call(
        body,
        grid=(NP // B,),
        in_specs=[
            pl.BlockSpec((B, CP), lambda i: (i, 0)),
            pl.BlockSpec((B, CP), lambda i: (i, 0)),
            pl.BlockSpec((B, 1), lambda i: (i, 0)),
            pl.BlockSpec((B, CP), lambda i: (i, 0)),
        ],
        out_specs=pl.BlockSpec((B, CP), lambda i: (i, 0)),
        out_shape=jax.ShapeDtypeStruct((NP, CP), jnp.float32),
    )(q0, q1, inv, hr)


def kernel(x, edge_index, W_l0, W_r0, b0, W_l1, W_r1, b1):
    pad = EPWP - EPW
    src = jnp.concatenate(
        [edge_index[0].reshape(NW, EPW), jnp.zeros((NW, pad), jnp.int32)], axis=1
    ).reshape(NW * NG, GRP, CH)
    dst = jnp.concatenate(
        [edge_index[1].reshape(NW, EPW), jnp.full((NW, pad), N, jnp.int32)], axis=1
    ).reshape(NW * NG, GRP, CH)

    zfeat = jnp.zeros((NP, HID), jnp.float32)
    zfeat2 = jnp.zeros((NP, CP), jnp.float32)
    zdeg = jnp.zeros((NP,), jnp.float32)
    ones = jnp.ones((CH,), jnp.float32)

    wl1p = jnp.pad(W_l1, ((0, 0), (0, CP - C)))
    wr1p = jnp.pad(W_r1, ((0, 0), (0, CP - C)))
    b1p = jnp.pad(b1, (0, CP - C)).reshape(1, CP)

    xp = jnp.pad(x, ((0, NP - N), (0, 0)))
    xt, xr = _tc_pre(xp, W_l0, W_r0)
    parts, degp = _sc_aggregate(HID, True)(xt, src, dst, zfeat, zdeg, ones)

    ht, hr, inv = _tc_mid(
        parts[0], parts[1],
        degp[0].reshape(NP, 1), degp[1].reshape(NP, 1),
        xr, b0.reshape(1, HID), wl1p, wr1p, b1p,
    )

    parts2 = _sc_aggregate(CP, False, tc_tiling=False)(ht, src, dst, zfeat2)[0]
    out = _tc_post(parts2[0], parts2[1], inv, hr)
    return out[:N, :C]


# speedup vs baseline: 1.3046x; 1.3046x over previous
import functools

import jax
import jax.numpy as jnp
from jax import lax
from jax.experimental import pallas as pl
from jax.experimental.pallas import tpu as pltpu
from jax.experimental.pallas import tpu_sc as plsc

N = 10000
NP = 10112
E = 320000
F_IN = 128
HID = 128
C = 40
CP = 48

NC = 2
NS = 16
NW = NC * NS
EPW = E // NW
CH = 128
GRP = 8
NG = 10
ITERS = NG * GRP
EPWP = ITERS * CH
RPT = NP // NS


def _sc_aggregate(D, with_deg, tc_tiling):
    out_type = [jax.ShapeDtypeStruct((NC, NP, D), jnp.float32)]
    scratch = [
        pltpu.VMEM((2, 2, GRP, CH), jnp.int32),
        pltpu.VMEM((2, CH, D), jnp.float32),
        pltpu.VMEM_SHARED((NP, D), jnp.float32),
        pltpu.SemaphoreType.DMA,
        pltpu.SemaphoreType.DMA,
        pltpu.SemaphoreType.DMA,
    ]
    if with_deg:
        out_type.append(jax.ShapeDtypeStruct((NC, NP), jnp.float32))
        scratch += [
            pltpu.VMEM((CH,), jnp.float32),
            pltpu.VMEM_SHARED((NP,), jnp.float32),
        ]

    mesh = plsc.VectorSubcoreMesh(
        core_axis_name="c", subcore_axis_name="s", num_cores=NC, num_subcores=NS
    )
    cp = pltpu.CompilerParams(use_tc_tiling_on_sc=tc_tiling)

    @functools.partial(pl.kernel, out_type=out_type, mesh=mesh, scratch_types=scratch,
                       compiler_params=cp)
    def agg(*refs):
        if with_deg:
            (table, srcw, dstw, zfeat, zdeg, onesh,
             part_out, deg_out,
             idxb, rows, acc, semr0, semr1, semi, ones_v, dacc) = refs
        else:
            (table, srcw, dstw, zfeat,
             part_out,
             idxb, rows, acc, semr0, semr1, semi) = refs

        c = lax.axis_index("c")
        s = lax.axis_index("s")
        w = s * NC + c
        blk0 = w * NG
        semr = (semr0, semr1)

        pltpu.sync_copy(srcw.at[blk0], idxb.at[0, 0])
        pltpu.sync_copy(dstw.at[blk0], idxb.at[0, 1])
        pltpu.async_copy(srcw.at[blk0 + 1], idxb.at[1, 0], semi)
        pltpu.async_copy(dstw.at[blk0 + 1], idxb.at[1, 1], semi)
        pltpu.async_copy(table.at[idxb.at[0, 0, 0]], rows.at[0], semr0)
        pltpu.async_copy(table.at[idxb.at[0, 0, 1]], rows.at[1], semr1)

        pltpu.sync_copy(zfeat.at[pl.ds(s * RPT, RPT)], acc.at[pl.ds(s * RPT, RPT)])
        if with_deg:
            pltpu.sync_copy(onesh, ones_v)

            @pl.when(s == 0)
            def _():
                pltpu.sync_copy(zdeg, dacc)

        plsc.subcore_barrier()

        def group(g, carry):
            half = lax.rem(g, 2)
            oh = 1 - half
            for j in range(GRP):
                b = j % 2
                pltpu.make_async_copy(table.at[idxb.at[half, 0, j]], rows.at[b],
                                      semr[b]).wait()
                pltpu.sync_copy(rows.at[b], acc.at[idxb.at[half, 1, j]], add=True)
                if with_deg:
                    pltpu.sync_copy(ones_v, dacc.at[idxb.at[half, 1, j]], add=True)
                if j < GRP - 2:
                    pltpu.async_copy(table.at[idxb.at[half, 0, j + 2]], rows.at[b],
                                     semr[b])
                else:
                    if j == GRP - 2:
                        @pl.when(g < NG - 1)
                        def _():
                            pltpu.make_async_copy(srcw.at[blk0 + g + 1],
                                                  idxb.at[oh, 0], semi).wait()
                            pltpu.make_async_copy(dstw.at[blk0 + g + 1],
                                                  idxb.at[oh, 1], semi).wait()

                    @pl.when(g < NG - 1)
                    def _():
                        pltpu.async_copy(table.at[idxb.at[oh, 0, j + 2 - GRP]],
                                         rows.at[b], semr[b])

            @pl.when(g < NG - 2)
            def _():
                pltpu.async_copy(srcw.at[blk0 + g + 2], idxb.at[half, 0], semi)
                pltpu.async_copy(dstw.at[blk0 + g + 2], idxb.at[half, 1], semi)
            return carry

        lax.fori_loop(0, NG, group, 0)
        plsc.subcore_barrier()

        pltpu.sync_copy(acc.at[pl.ds(s * RPT, RPT)], part_out.at[c, pl.ds(s * RPT, RPT)])
        if with_deg:
            @pl.when(s == 0)
            def _():
                pltpu.sync_copy(dacc, deg_out.at[c])

    return agg


def _tc_pre(x, wl, wr):
    B = 1264

    def body(x_ref, wl_ref, wr_ref, xt_ref, xr_ref):
        xb = x_ref[...]
        xt_ref[...] = jnp.dot(xb, wl_ref[...], preferred_element_type=jnp.float32)
        xr_ref[...] = jnp.dot(xb, wr_ref[...], preferred_element_type=jnp.float32)

    return pl.pallas_call(
        body,
        grid=(NP // B,),
        in_specs=[
            pl.BlockSpec((B, F_IN), lambda i: (i, 0)),
            pl.BlockSpec((F_IN, HID), lambda i: (0, 0)),
            pl.BlockSpec((F_IN, HID), lambda i: (0, 0)),
        ],
        out_specs=[pl.BlockSpec((B, HID), lambda i: (i, 0))] * 2,
        out_shape=[jax.ShapeDtypeStruct((NP, HID), jnp.float32)] * 2,
    )(x, wl, wr)


def _tc_mid(p0, p1, d0, d1, xr, b0, wl1, wr1, b1):
    B = 1264

    def body(p0_ref, p1_ref, d0_ref, d1_ref, xr_ref, b0_ref, wl_ref, wr_ref, b1_ref,
             ht_ref, hr_ref, inv_ref):
        inv = 1.0 / jnp.maximum(d0_ref[...] + d1_ref[...], 1.0)
        h = jnp.maximum((p0_ref[...] + p1_ref[...]) * inv + b0_ref[...] + xr_ref[...], 0.0)
        ht_ref[...] = jnp.dot(h, wl_ref[...], preferred_element_type=jnp.float32)
        hr_ref[...] = jnp.dot(h, wr_ref[...], preferred_element_type=jnp.float32) + b1_ref[...]
        inv_ref[...] = inv

    return pl.pallas_call(
        body,
        grid=(NP // B,),
        in_specs=[
            pl.BlockSpec((B, HID), lambda i: (i, 0)),
            pl.BlockSpec((B, HID), lambda i: (i, 0)),
            pl.BlockSpec((B, 1), lambda i: (i, 0)),
            pl.BlockSpec((B, 1), lambda i: (i, 0)),
            pl.BlockSpec((B, HID), lambda i: (i, 0)),
            pl.BlockSpec((1, HID), lambda i: (0, 0)),
            pl.BlockSpec((HID, CP), lambda i: (0, 0)),
            pl.BlockSpec((HID, CP), lambda i: (0, 0)),
            pl.BlockSpec((1, CP), lambda i: (0, 0)),
        ],
        out_specs=[
            pl.BlockSpec((B, CP), lambda i: (i, 0)),
            pl.BlockSpec((B, CP), lambda i: (i, 0)),
            pl.BlockSpec((B, 1), lambda i: (i, 0)),
        ],
        out_shape=[
            jax.ShapeDtypeStruct((NP, CP), jnp.float32),
            jax.ShapeDtypeStruct((NP, CP), jnp.float32),
            jax.ShapeDtypeStruct((NP, 1), jnp.float32),
        ],
    )(p0, p1, d0, d1, xr, b0, wl1, wr1, b1)


def _tc_post(q0, q1, inv, hr):
    B = 1264

    def body(q0_ref, q1_ref, inv_ref, hr_ref, o_ref):
        o_ref[...] = (q0_ref[...] + q1_ref[...]) * inv_ref[...] + hr_ref[...]

    return pl.pallas_call(
        body,
        grid=(NP // B,),
        in_specs=[
            pl.BlockSpec((B, CP), lambda i: (i, 0)),
            pl.BlockSpec((B, CP), lambda i: (i, 0)),
            pl.BlockSpec((B, 1), lambda i: (i, 0)),
            pl.BlockSpec((B, CP), lambda i: (i, 0)),
        ],
        out_specs=pl.BlockSpec((B, CP), lambda i: (i, 0)),
        out_shape=jax.ShapeDtypeStruct((NP, CP), jnp.float32),
    )(q0, q1, inv, hr)


def kernel(x, edge_index, W_l0, W_r0, b0, W_l1, W_r1, b1):
    pad = EPWP - EPW
    src = jnp.concatenate(
        [edge_index[0].reshape(NW, EPW), jnp.zeros((NW, pad), jnp.int32)], axis=1
    ).reshape(NW * NG, GRP, CH)
    padrows = N + jnp.arange(pad, dtype=jnp.int32) % (NP - N)
    dst = jnp.concatenate(
        [edge_index[1].reshape(NW, EPW), jnp.tile(padrows, (NW, 1))], axis=1
    ).reshape(NW * NG, GRP, CH)

    zfeat = jnp.zeros((NP, HID), jnp.float32)
    zfeat2 = jnp.zeros((NP, CP), jnp.float32)
    zdeg = jnp.zeros((NP,), jnp.float32)
    ones = jnp.ones((CH,), jnp.float32)

    wl1p = jnp.pad(W_l1, ((0, 0), (0, CP - C)))
    wr1p = jnp.pad(W_r1, ((0, 0), (0, CP - C)))
    b1p = jnp.pad(b1, (0, CP - C)).reshape(1, CP)

    xp = jnp.pad(x, ((0, NP - N), (0, 0)))
    xt, xr = _tc_pre(xp, W_l0, W_r0)
    parts, degp = _sc_aggregate(HID, True, True)(xt, src, dst, zfeat, zdeg, ones)

    ht, hr, inv = _tc_mid(
        parts[0], parts[1],
        degp[0].reshape(NP, 1), degp[1].reshape(NP, 1),
        xr, b0.reshape(1, HID), wl1p, wr1p, b1p,
    )

    parts2 = _sc_aggregate(CP, False, False)(ht, src, dst, zfeat2)[0]
    out = _tc_post(parts2[0], parts2[1], inv, hr)
    return out[:N, :C]


# speedup vs baseline: 1.3061x; 1.0012x over previous
import functools

import jax
import jax.numpy as jnp
from jax import lax
from jax.experimental import pallas as pl
from jax.experimental.pallas import tpu as pltpu
from jax.experimental.pallas import tpu_sc as plsc

N = 10000
NP = 10112
E = 320000
F_IN = 128
HID = 128
C = 40
CP = 48

NC = 2
NS = 16
NW = NC * NS
EPW = E // NW
CH = 128
GRP = 8
NG = 10
ITERS = NG * GRP
EPWP = ITERS * CH
RPT = NP // NS


def _sc_aggregate(D, with_deg, tc_tiling):
    out_type = [jax.ShapeDtypeStruct((NC, NP, D), jnp.float32)]
    scratch = [
        pltpu.VMEM((2, 2, GRP, CH), jnp.int32),
        pltpu.VMEM((2, CH, D), jnp.float32),
        pltpu.VMEM_SHARED((NP, D), jnp.float32),
        pltpu.SemaphoreType.DMA,
        pltpu.SemaphoreType.DMA,
        pltpu.SemaphoreType.DMA,
        pltpu.SemaphoreType.DMA,
    ]
    if with_deg:
        out_type.append(jax.ShapeDtypeStruct((NC, NP), jnp.float32))
        scratch += [
            pltpu.VMEM((CH,), jnp.float32),
            pltpu.VMEM_SHARED((NP,), jnp.float32),
        ]

    mesh = plsc.VectorSubcoreMesh(
        core_axis_name="c", subcore_axis_name="s", num_cores=NC, num_subcores=NS
    )
    cp = pltpu.CompilerParams(use_tc_tiling_on_sc=tc_tiling)

    @functools.partial(pl.kernel, out_type=out_type, mesh=mesh, scratch_types=scratch,
                       compiler_params=cp)
    def agg(*refs):
        if with_deg:
            (table, srcw, dstw, zfeat, zdeg, onesh,
             part_out, deg_out,
             idxb, rows, acc, semr0, semr1, semi, semd, ones_v, dacc) = refs
        else:
            (table, srcw, dstw, zfeat,
             part_out,
             idxb, rows, acc, semr0, semr1, semi, semd) = refs

        c = lax.axis_index("c")
        s = lax.axis_index("s")
        w = s * NC + c
        blk0 = w * NG
        semr = (semr0, semr1)

        pltpu.sync_copy(srcw.at[blk0], idxb.at[0, 0])
        pltpu.sync_copy(dstw.at[blk0], idxb.at[0, 1])
        pltpu.async_copy(srcw.at[blk0 + 1], idxb.at[1, 0], semi)
        pltpu.async_copy(dstw.at[blk0 + 1], idxb.at[1, 1], semi)
        pltpu.async_copy(table.at[idxb.at[0, 0, 0]], rows.at[0], semr0)
        pltpu.async_copy(table.at[idxb.at[0, 0, 1]], rows.at[1], semr1)

        pltpu.sync_copy(zfeat.at[pl.ds(s * RPT, RPT)], acc.at[pl.ds(s * RPT, RPT)])
        if with_deg:
            pltpu.sync_copy(onesh, ones_v)

            @pl.when(s == 0)
            def _():
                pltpu.sync_copy(zdeg, dacc)

        plsc.subcore_barrier()

        def group(g, carry):
            half = lax.rem(g, 2)
            oh = 1 - half
            for j in range(GRP):
                b = j % 2
                pltpu.make_async_copy(table.at[idxb.at[half, 0, j]], rows.at[b],
                                      semr[b]).wait()
                pltpu.sync_copy(rows.at[b], acc.at[idxb.at[half, 1, j]], add=True)
                if with_deg:
                    if j > 0:
                        pltpu.make_async_copy(ones_v, dacc.at[idxb.at[half, 1, j - 1]],
                                              semd).wait()
                    pltpu.async_copy(ones_v, dacc.at[idxb.at[half, 1, j]], semd, add=True)
                if j < GRP - 2:
                    pltpu.async_copy(table.at[idxb.at[half, 0, j + 2]], rows.at[b],
                                     semr[b])
                else:
                    if j == GRP - 2:
                        @pl.when(g < NG - 1)
                        def _():
                            pltpu.make_async_copy(srcw.at[blk0 + g + 1],
                                                  idxb.at[oh, 0], semi).wait()
                            pltpu.make_async_copy(dstw.at[blk0 + g + 1],
                                                  idxb.at[oh, 1], semi).wait()

                    @pl.when(g < NG - 1)
                    def _():
                        pltpu.async_copy(table.at[idxb.at[oh, 0, j + 2 - GRP]],
                                         rows.at[b], semr[b])

            if with_deg:
                pltpu.make_async_copy(ones_v, dacc.at[idxb.at[half, 1, GRP - 1]],
                                      semd).wait()

            @pl.when(g < NG - 2)
            def _():
                pltpu.async_copy(srcw.at[blk0 + g + 2], idxb.at[half, 0], semi)
                pltpu.async_copy(dstw.at[blk0 + g + 2], idxb.at[half, 1], semi)
            return carry

        lax.fori_loop(0, NG, group, 0)
        plsc.subcore_barrier()

        pltpu.sync_copy(acc.at[pl.ds(s * RPT, RPT)], part_out.at[c, pl.ds(s * RPT, RPT)])
        if with_deg:
            @pl.when(s == 0)
            def _():
                pltpu.sync_copy(dacc, deg_out.at[c])

    return agg


def _tc_pre(x, wl, wr):
    B = 1264

    def body(x_ref, wl_ref, wr_ref, xt_ref, xr_ref):
        xb = x_ref[...]
        xt_ref[...] = jnp.dot(xb, wl_ref[...], preferred_element_type=jnp.float32)
        xr_ref[...] = jnp.dot(xb, wr_ref[...], preferred_element_type=jnp.float32)

    return pl.pallas_call(
        body,
        grid=(NP // B,),
        in_specs=[
            pl.BlockSpec((B, F_IN), lambda i: (i, 0)),
            pl.BlockSpec((F_IN, HID), lambda i: (0, 0)),
            pl.BlockSpec((F_IN, HID), lambda i: (0, 0)),
        ],
        out_specs=[pl.BlockSpec((B, HID), lambda i: (i, 0))] * 2,
        out_shape=[jax.ShapeDtypeStruct((NP, HID), jnp.float32)] * 2,
    )(x, wl, wr)


def _tc_mid(p0, p1, d0, d1, xr, b0, wl1, wr1, b1):
    B = 1264

    def body(p0_ref, p1_ref, d0_ref, d1_ref, xr_ref, b0_ref, wl_ref, wr_ref, b1_ref,
             ht_ref, hr_ref, inv_ref):
        inv = 1.0 / jnp.maximum(d0_ref[...] + d1_ref[...], 1.0)
        h = jnp.maximum((p0_ref[...] + p1_ref[...]) * inv + b0_ref[...] + xr_ref[...], 0.0)
        ht_ref[...] = jnp.dot(h, wl_ref[...], preferred_element_type=jnp.float32)
        hr_ref[...] = jnp.dot(h, wr_ref[...], preferred_element_type=jnp.float32) + b1_ref[...]
        inv_ref[...] = inv

    return pl.pallas_call(
        body,
        grid=(NP // B,),
        in_specs=[
            pl.BlockSpec((B, HID), lambda i: (i, 0)),
            pl.BlockSpec((B, HID), lambda i: (i, 0)),
            pl.BlockSpec((B, 1), lambda i: (i, 0)),
            pl.BlockSpec((B, 1), lambda i: (i, 0)),
            pl.BlockSpec((B, HID), lambda i: (i, 0)),
            pl.BlockSpec((1, HID), lambda i: (0, 0)),
            pl.BlockSpec((HID, CP), lambda i: (0, 0)),
            pl.BlockSpec((HID, CP), lambda i: (0, 0)),
            pl.BlockSpec((1, CP), lambda i: (0, 0)),
        ],
        out_specs=[
            pl.BlockSpec((B, CP), lambda i: (i, 0)),
            pl.BlockSpec((B, CP), lambda i: (i, 0)),
            pl.BlockSpec((B, 1), lambda i: (i, 0)),
        ],
        out_shape=[
            jax.ShapeDtypeStruct((NP, CP), jnp.float32),
            jax.ShapeDtypeStruct((NP, CP), jnp.float32),
            jax.ShapeDtypeStruct((NP, 1), jnp.float32),
        ],
    )(p0, p1, d0, d1, xr, b0, wl1, wr1, b1)


def _tc_post(q0, q1, inv, hr):
    B = 1264

    def body(q0_ref, q1_ref, inv_ref, hr_ref, o_ref):
        o_ref[...] = (q0_ref[...] + q1_ref[...]) * inv_ref[...] + hr_ref[...]

    return pl.pallas_call(
        body,
        grid=(NP // B,),
        in_specs=[
            pl.BlockSpec((B, CP), lambda i: (i, 0)),
            pl.BlockSpec((B, CP), lambda i: (i, 0)),
            pl.BlockSpec((B, 1), lambda i: (i, 0)),
            pl.BlockSpec((B, CP), lambda i: (i, 0)),
        ],
        out_specs=pl.BlockSpec((B, CP), lambda i: (i, 0)),
        out_shape=jax.ShapeDtypeStruct((NP, CP), jnp.float32),
    )(q0, q1, inv, hr)


def kernel(x, edge_index, W_l0, W_r0, b0, W_l1, W_r1, b1):
    pad = EPWP - EPW
    src = jnp.concatenate(
        [edge_index[0].reshape(NW, EPW), jnp.zeros((NW, pad), jnp.int32)], axis=1
    ).reshape(NW * NG, GRP, CH)
    padrows = N + jnp.arange(pad, dtype=jnp.int32) % (NP - N)
    dst = jnp.concatenate(
        [edge_index[1].reshape(NW, EPW), jnp.tile(padrows, (NW, 1))], axis=1
    ).reshape(NW * NG, GRP, CH)

    zfeat = jnp.zeros((NP, HID), jnp.float32)
    zfeat2 = jnp.zeros((NP, CP), jnp.float32)
    zdeg = jnp.zeros((NP,), jnp.float32)
    ones = jnp.ones((CH,), jnp.float32)

    wl1p = jnp.pad(W_l1, ((0, 0), (0, CP - C)))
    wr1p = jnp.pad(W_r1, ((0, 0), (0, CP - C)))
    b1p = jnp.pad(b1, (0, CP - C)).reshape(1, CP)

    xp = jnp.pad(x, ((0, NP - N), (0, 0)))
    xt, xr = _tc_pre(xp, W_l0, W_r0)
    parts, degp = _sc_aggregate(HID, True, True)(xt, src, dst, zfeat, zdeg, ones)

    ht, hr, inv = _tc_mid(
        parts[0], parts[1],
        degp[0].reshape(NP, 1), degp[1].reshape(NP, 1),
        xr, b0.reshape(1, HID), wl1p, wr1p, b1p,
    )

    parts2 = _sc_aggregate(CP, False, False)(ht, src, dst, zfeat2)[0]
    out = _tc_post(parts2[0], parts2[1], inv, hr)
    return out[:N, :C]
